# baseline (speedup 1.0000x reference)
import functools

import jax
import jax.numpy as jnp
from jax import lax
from jax.experimental import pallas as pl
from jax.experimental.pallas import tpu as pltpu
from jax.experimental.pallas import tpu_sc as plsc

B, C, N, L = 8, 2, 256, 4000
M = 128
Q = L // 4
FS = [96] * 10 + [40]
FMAX = max(FS)
OUT_LEN = M * (L + 1) - 1


def _sc_body(x_hbm, xw_hbm, out_hbm, xb0, xb1, wb, ob0, ob1, hb0, hb1, hw, tb):
    cid = lax.axis_index("c")
    sid = lax.axis_index("s")
    wid = sid * 2 + cid
    b = wid // 4
    q = wid % 4
    ks = pl.multiple_of(q * Q, 8)
    iota = lax.iota(jnp.int32, 16)
    zero = jnp.zeros((16,), jnp.float32)

    carry0 = [zero] * 8
    carry1 = [zero] * 8
    hcol = jnp.full((16,), 7, jnp.int32)
    hstart = pl.multiple_of(jnp.maximum(ks - 8, 0), 8)
    qmask = jnp.full((16,), jnp.where(q > 0, 1.0, 0.0), jnp.float32)

    pltpu.sync_copy(x_hbm.at[b, 0, pl.ds(M, M), pl.ds(hstart, 8)], hb0)
    pltpu.sync_copy(x_hbm.at[b, 1, pl.ds(M, M), pl.ds(hstart, 8)], hb1)
    pltpu.sync_copy(xw_hbm.at[b, pl.ds(M, M), pl.ds(hstart, 8)], hw)

    for g in range(8):
        rows = g * 16 + iota
        wv = plsc.load_gather(hw, [rows, hcol]) * qmask
        carry0[g] = plsc.load_gather(hb0, [rows, hcol]) * wv
        carry1[g] = plsc.load_gather(hb1, [rows, hcol]) * wv

    carry = tuple(carry0) + tuple(carry1)

    k0 = ks
    for f in FS:
        k0 = pl.multiple_of(k0, 8)
        pltpu.sync_copy(x_hbm.at[b, 0, :, pl.ds(k0, f)], xb0.at[:, pl.ds(0, f)])
        pltpu.sync_copy(x_hbm.at[b, 1, :, pl.ds(k0, f)], xb1.at[:, pl.ds(0, f)])
        pltpu.sync_copy(xw_hbm.at[b, :, pl.ds(k0, f)], wb.at[:, pl.ds(0, f)])

        def body(j, cr, _f=f):
            col = jnp.full((16,), j, jnp.int32)
            new = list(cr)
            base = j * M
            for g in range(8):
                rl = g * 16 + iota
                ru = rl + M
                wl = plsc.load_gather(wb, [rl, col])
                wu = plsc.load_gather(wb, [ru, col])
                ob0[pl.ds(base + g * 16, 16)] = (
                    plsc.load_gather(xb0, [rl, col]) * wl + cr[g])
                ob1[pl.ds(base + g * 16, 16)] = (
                    plsc.load_gather(xb1, [rl, col]) * wl + cr[8 + g])
                new[g] = plsc.load_gather(xb0, [ru, col]) * wu
                new[8 + g] = plsc.load_gather(xb1, [ru, col]) * wu
            return tuple(new)

        carry = lax.fori_loop(0, f, body, carry)

        pltpu.sync_copy(ob0.at[pl.ds(0, f * M)], out_hbm.at[b, 0, pl.ds(k0 * M, f * M)])
        pltpu.sync_copy(ob1.at[pl.ds(0, f * M)], out_hbm.at[b, 1, pl.ds(k0 * M, f * M)])
        k0 += f

    @pl.when(q == 3)
    def _tail():
        for c, cbase in ((0, 0), (1, 8)):
            for g in range(8):
                tb[pl.ds(g * 16, 16)] = carry[cbase + g]
            pltpu.sync_copy(tb.at[pl.ds(0, M - 1)],
                            out_hbm.at[b, c, pl.ds(L * M, M - 1)])


@functools.lru_cache(maxsize=1)
def _oadd():
    return pl.kernel(
        _sc_body,
        out_type=jax.ShapeDtypeStruct((B, C, OUT_LEN), jnp.float32),
        mesh=plsc.VectorSubcoreMesh(core_axis_name="c", subcore_axis_name="s"),
        scratch_types=(
            [pltpu.VMEM((N, FMAX), jnp.float32)] * 3
            + [pltpu.VMEM((FMAX * M,), jnp.float32)] * 2
            + [pltpu.VMEM((M, 8), jnp.float32)] * 3
            + [pltpu.VMEM((M,), jnp.float32)]
        ),
        compiler_params=pltpu.CompilerParams(use_tc_tiling_on_sc=False,
                                             needs_layout_passes=False),
    )


def kernel(x, x_wave, encoder_padding):
    del encoder_padding
    return _oadd()(x, x_wave)


# speedup vs baseline: 1.5530x; 1.5530x over previous
import functools

import jax
import jax.numpy as jnp
from jax import lax
from jax.experimental import pallas as pl
from jax.experimental.pallas import tpu as pltpu
from jax.experimental.pallas import tpu_sc as plsc

B, C, N, L = 8, 2, 256, 4000
M = 128
Q = L // 4
FS = [96] * 10 + [40]
FMAX = max(FS)
H = 8
OUT_LEN = M * (L + 1) - 1


def _sc_body(x_hbm, xw_hbm, out_hbm, xb0, xb1, wb, ob0, ob1, tb):
    cid = lax.axis_index("c")
    sid = lax.axis_index("s")
    wid = sid * 2 + cid
    b = wid // 4
    q = wid % 4
    ks = pl.multiple_of(q * Q, 8)
    iota = lax.iota(jnp.int32, 16)
    zero = jnp.zeros((16,), jnp.float32)

    k0 = ks
    for t, f in enumerate(FS):
        k0 = pl.multiple_of(k0, 8)
        hstart = pl.multiple_of(jnp.maximum(k0 - H, 0), 8)
        pltpu.sync_copy(x_hbm.at[b, 0, :, pl.ds(hstart, H)], xb0.at[:, pl.ds(0, H)])
        pltpu.sync_copy(x_hbm.at[b, 1, :, pl.ds(hstart, H)], xb1.at[:, pl.ds(0, H)])
        pltpu.sync_copy(xw_hbm.at[b, :, pl.ds(hstart, H)], wb.at[:, pl.ds(0, H)])
        pltpu.sync_copy(x_hbm.at[b, 0, :, pl.ds(k0, f)], xb0.at[:, pl.ds(H, f)])
        pltpu.sync_copy(x_hbm.at[b, 1, :, pl.ds(k0, f)], xb1.at[:, pl.ds(H, f)])
        pltpu.sync_copy(xw_hbm.at[b, :, pl.ds(k0, f)], wb.at[:, pl.ds(H, f)])

        if t == 0:
            @pl.when(q == 0)
            def _zero_halo():
                hc = jnp.full((16,), H - 1, jnp.int32)
                for g in range(16):
                    plsc.store_scatter(wb, [g * 16 + iota, hc], zero)

        def body(j, _):
            cu = jnp.full((16,), j, jnp.int32) + (H - 1)
            cl = cu + 1
            base = j * M
            for g in range(8):
                rl = g * 16 + iota
                ru = rl + M
                wl = plsc.load_gather(wb, [rl, cl])
                wu = plsc.load_gather(wb, [ru, cu])
                u0 = plsc.load_gather(xb0, [ru, cu]) * wu
                u1 = plsc.load_gather(xb1, [ru, cu]) * wu
                ob0[pl.ds(base + g * 16, 16)] = (
                    plsc.load_gather(xb0, [rl, cl]) * wl + u0)
                ob1[pl.ds(base + g * 16, 16)] = (
                    plsc.load_gather(xb1, [rl, cl]) * wl + u1)
            return 0

        lax.fori_loop(0, f, body, 0)

        pltpu.sync_copy(ob0.at[pl.ds(0, f * M)], out_hbm.at[b, 0, pl.ds(k0 * M, f * M)])
        pltpu.sync_copy(ob1.at[pl.ds(0, f * M)], out_hbm.at[b, 1, pl.ds(k0 * M, f * M)])
        k0 += f

    @pl.when(q == 3)
    def _tail():
        lc = jnp.full((16,), H + FS[-1] - 1, jnp.int32)
        for c, xb in ((0, xb0), (1, xb1)):
            for g in range(8):
                ru = g * 16 + iota + M
                wu = plsc.load_gather(wb, [ru, lc])
                tb[pl.ds(g * 16, 16)] = plsc.load_gather(xb, [ru, lc]) * wu
            pltpu.sync_copy(tb.at[pl.ds(0, M - 1)],
                            out_hbm.at[b, c, pl.ds(L * M, M - 1)])


@functools.lru_cache(maxsize=1)
def _oadd():
    return pl.kernel(
        _sc_body,
        out_type=jax.ShapeDtypeStruct((B, C, OUT_LEN), jnp.float32),
        mesh=plsc.VectorSubcoreMesh(core_axis_name="c", subcore_axis_name="s"),
        scratch_types=(
            [pltpu.VMEM((N, FMAX + H), jnp.float32)] * 3
            + [pltpu.VMEM((FMAX * M,), jnp.float32)] * 2
            + [pltpu.VMEM((M,), jnp.float32)]
        ),
        compiler_params=pltpu.CompilerParams(use_tc_tiling_on_sc=False,
                                             needs_layout_passes=False),
    )


def kernel(x, x_wave, encoder_padding):
    del encoder_padding
    return _oadd()(x, x_wave)


# speedup vs baseline: 1.8587x; 1.1969x over previous
import functools

import jax
import jax.numpy as jnp
from jax import lax
from jax.experimental import pallas as pl
from jax.experimental.pallas import tpu as pltpu
from jax.experimental.pallas import tpu_sc as plsc

B, C, N, L = 8, 2, 256, 4000
M = 128
Q = L // 4
FS = [96] * 10 + [40]
FMAX = max(FS)
H = 8
OUT_LEN = M * (L + 1) - 1


def _sc_body(x_hbm, xw_hbm, out_hbm, xb0, xb1, wb, ob0, ob1, tb):
    cid = lax.axis_index("c")
    sid = lax.axis_index("s")
    wid = sid * 2 + cid
    b = wid // 4
    q = wid % 4
    ks = pl.multiple_of(q * Q, 8)
    iota = lax.iota(jnp.int32, 16)
    zero = jnp.zeros((16,), jnp.float32)

    k0 = ks
    for t, f in enumerate(FS):
        k0 = pl.multiple_of(k0, 8)
        hstart = pl.multiple_of(jnp.maximum(k0 - H, 0), 8)
        pltpu.sync_copy(x_hbm.at[b, 0, :, pl.ds(hstart, H)], xb0.at[:, pl.ds(0, H)])
        pltpu.sync_copy(x_hbm.at[b, 1, :, pl.ds(hstart, H)], xb1.at[:, pl.ds(0, H)])
        pltpu.sync_copy(xw_hbm.at[b, :, pl.ds(hstart, H)], wb.at[:, pl.ds(0, H)])
        pltpu.sync_copy(x_hbm.at[b, 0, :, pl.ds(k0, f)], xb0.at[:, pl.ds(H, f)])
        pltpu.sync_copy(x_hbm.at[b, 1, :, pl.ds(k0, f)], xb1.at[:, pl.ds(H, f)])
        pltpu.sync_copy(xw_hbm.at[b, :, pl.ds(k0, f)], wb.at[:, pl.ds(H, f)])

        if t == 0:
            @pl.when(q == 0)
            def _zero_halo():
                hc = jnp.full((16,), H - 1, jnp.int32)
                for g in range(16):
                    plsc.store_scatter(wb, [g * 16 + iota, hc], zero)

        @plsc.parallel_loop(0, f, unroll=4)
        def _frames(j):
            cu = jnp.full((16,), j, jnp.int32) + (H - 1)
            cl = cu + 1
            base = j * M
            for g in range(8):
                rl = g * 16 + iota
                ru = rl + M
                wl = plsc.load_gather(wb, [rl, cl])
                wu = plsc.load_gather(wb, [ru, cu])
                u0 = plsc.load_gather(xb0, [ru, cu]) * wu
                u1 = plsc.load_gather(xb1, [ru, cu]) * wu
                ob0[pl.ds(base + g * 16, 16)] = (
                    plsc.load_gather(xb0, [rl, cl]) * wl + u0)
                ob1[pl.ds(base + g * 16, 16)] = (
                    plsc.load_gather(xb1, [rl, cl]) * wl + u1)

        pltpu.sync_copy(ob0.at[pl.ds(0, f * M)], out_hbm.at[b, 0, pl.ds(k0 * M, f * M)])
        pltpu.sync_copy(ob1.at[pl.ds(0, f * M)], out_hbm.at[b, 1, pl.ds(k0 * M, f * M)])
        k0 += f

    @pl.when(q == 3)
    def _tail():
        lc = jnp.full((16,), H + FS[-1] - 1, jnp.int32)
        for c, xb in ((0, xb0), (1, xb1)):
            for g in range(8):
                ru = g * 16 + iota + M
                wu = plsc.load_gather(wb, [ru, lc])
                tb[pl.ds(g * 16, 16)] = plsc.load_gather(xb, [ru, lc]) * wu
            pltpu.sync_copy(tb.at[pl.ds(0, M - 1)],
                            out_hbm.at[b, c, pl.ds(L * M, M - 1)])


@functools.lru_cache(maxsize=1)
def _oadd():
    return pl.kernel(
        _sc_body,
        out_type=jax.ShapeDtypeStruct((B, C, OUT_LEN), jnp.float32),
        mesh=plsc.VectorSubcoreMesh(core_axis_name="c", subcore_axis_name="s"),
        scratch_types=(
            [pltpu.VMEM((N, FMAX + H), jnp.float32)] * 3
            + [pltpu.VMEM((FMAX * M,), jnp.float32)] * 2
            + [pltpu.VMEM((M,), jnp.float32)]
        ),
        compiler_params=pltpu.CompilerParams(use_tc_tiling_on_sc=False,
                                             needs_layout_passes=False),
    )


def kernel(x, x_wave, encoder_padding):
    del encoder_padding
    return _oadd()(x, x_wave)
